# baseline (speedup 1.0000x reference)
import functools

import jax
import jax.numpy as jnp
from jax import lax
from jax.experimental import pallas as pl
from jax.experimental.pallas import tpu as pltpu
from jax.experimental.pallas import tpu_sc as plsc

_CH = 64
_NC = 2
_NS = 16
_L = 16


@functools.lru_cache(maxsize=None)
def _build(V, D, B, S):
    N = B * S
    NW = _NC * _NS
    BPW = N // NW
    KJ = BPW // _CH
    WPB = S // BPW

    assert N % NW == 0 and BPW % _CH == 0 and S % BPW == 0

    mesh = plsc.VectorSubcoreMesh(core_axis_name="c", subcore_axis_name="s")

    @functools.partial(
        pl.kernel,
        mesh=mesh,
        out_type=jax.ShapeDtypeStruct((B, S, D), jnp.float32),
        scratch_types=(
            [pltpu.VMEM((BPW,), jnp.int32),
             pltpu.VMEM((KJ, _CH), jnp.int32),
             pltpu.VMEM((BPW, D), jnp.float32),
             pltpu.VMEM_SHARED((_NS * BPW, D), jnp.float32)]
            + [pltpu.SemaphoreType.DMA] * (2 * KJ + 3)
        ),
    )
    def embed(idx_hbm, tok_hbm, pos_hbm, out_hbm,
              idx_v, sidx_v, rows_v, acc_sh, *sems):
        gsems, asems = sems[:KJ], sems[KJ:2 * KJ]
        psem, isem, ssem = sems[2 * KJ:]
        sid = lax.axis_index("s")
        wid = sid * _NC + lax.axis_index("c")
        b = wid // WPB
        sbase = lax.rem(wid, WPB) * BPW
        abase = sid * BPW

        pcopy = pltpu.async_copy(pos_hbm.at[pl.ds(sbase, BPW)],
                                 acc_sh.at[pl.ds(abase, BPW)], psem)
        icopy = pltpu.async_copy(idx_hbm.at[b, pl.ds(sbase, BPW)], idx_v,
                                 isem)
        for j in range(KJ):
            for k in range(_CH // _L):
                sidx_v[j, pl.ds(k * _L, _L)] = (
                    lax.iota(jnp.int32, _L) + (abase + j * _CH + k * _L))
        icopy.wait()
        gathers = [
            pltpu.async_copy(tok_hbm.at[idx_v.at[pl.ds(j * _CH, _CH)]],
                             rows_v.at[pl.ds(j * _CH, _CH)], gsems[j])
            for j in range(KJ)
        ]
        pcopy.wait()
        adds = []
        for j in range(KJ):
            gathers[j].wait()
            adds.append(
                pltpu.async_copy(rows_v.at[pl.ds(j * _CH, _CH)],
                                 acc_sh.at[sidx_v.at[j]], asems[j],
                                 add=True))
        stores = []
        for j in range(KJ):
            adds[j].wait()
            stores.append(
                pltpu.async_copy(acc_sh.at[pl.ds(abase + j * _CH, _CH)],
                                 out_hbm.at[b, pl.ds(sbase + j * _CH, _CH)],
                                 ssem))
        for st in stores:
            st.wait()

    return embed


def kernel(x, token_table, pos_table):
    B, S = x.shape
    V, D = token_table.shape
    return _build(V, D, B, S)(x.astype(jnp.int32), token_table, pos_table)


# speedup vs baseline: 1.0471x; 1.0471x over previous
import functools

import jax
import jax.numpy as jnp
from jax import lax
from jax.experimental import pallas as pl
from jax.experimental.pallas import tpu as pltpu
from jax.experimental.pallas import tpu_sc as plsc

_CH = 64
_NC = 2
_NS = 16


@functools.lru_cache(maxsize=None)
def _build(V, D, B, S):
    N = B * S
    NW = _NC * _NS
    BPW = N // NW
    KJ = BPW // _CH
    WPB = S // BPW

    assert N % NW == 0 and BPW % _CH == 0 and S % BPW == 0

    mesh = plsc.VectorSubcoreMesh(core_axis_name="c", subcore_axis_name="s")

    @functools.partial(
        pl.kernel,
        mesh=mesh,
        out_type=jax.ShapeDtypeStruct((B, S, D), jnp.float32),
        scratch_types=(
            [pltpu.VMEM((BPW,), jnp.int32),
             pltpu.VMEM((BPW, D), jnp.float32)]
            + [pltpu.SemaphoreType.DMA] * (KJ + 2)
        ),
    )
    def embed(idx_hbm, tok_hbm, pos_hbm, out_hbm, idx_v, rows_v, *sems):
        psems, isem, ssem = sems[:KJ], sems[KJ], sems[KJ + 1]
        wid = lax.axis_index("s") * _NC + lax.axis_index("c")
        b = wid // WPB
        sbase = lax.rem(wid, WPB) * BPW

        icopy = pltpu.async_copy(idx_hbm.at[b, pl.ds(sbase, BPW)], idx_v,
                                 isem)
        prefills = [
            pltpu.async_copy(pos_hbm.at[pl.ds(sbase + j * _CH, _CH)],
                             rows_v.at[pl.ds(j * _CH, _CH)], psems[j])
            for j in range(KJ)
        ]
        icopy.wait()
        gathers = []
        for j in range(KJ):
            prefills[j].wait()
            gathers.append(
                pltpu.async_copy(tok_hbm.at[idx_v.at[pl.ds(j * _CH, _CH)]],
                                 rows_v.at[pl.ds(j * _CH, _CH)], psems[j],
                                 add=True))
        stores = []
        for j in range(KJ):
            gathers[j].wait()
            stores.append(
                pltpu.async_copy(rows_v.at[pl.ds(j * _CH, _CH)],
                                 out_hbm.at[b, pl.ds(sbase + j * _CH, _CH)],
                                 ssem))
        for st in stores:
            st.wait()

    return embed


def kernel(x, token_table, pos_table):
    B, S = x.shape
    V, D = token_table.shape
    return _build(V, D, B, S)(x.astype(jnp.int32), token_table, pos_table)
